# baseline (speedup 1.0000x reference)
import jax
import jax.numpy as jnp
from jax.experimental import pallas as pl
from jax.experimental.pallas import tpu as pltpu


_ROWS_PER_BLOCK = 1024


def _poly_body(imp_ref, c_ref, x_ref, o_ref):
    m = jax.nn.sigmoid(imp_ref[...])
    m = m / (jnp.mean(m) + 1e-6)
    c0, c1, c2 = c_ref[0], c_ref[1], c_ref[2]
    a = c0 * m
    b = c1 * (m * m)
    g = c2 * (m * m * m)
    x = x_ref[...]
    o_ref[...] = x * (a + x * (b + x * g))


def kernel(x, coeffs, importance):
    B, T, D = x.shape
    rows = B * T
    x2 = x.reshape(rows, D)
    imp2 = importance.reshape(1, D)
    r = _ROWS_PER_BLOCK
    out = pl.pallas_call(
        _poly_body,
        grid=(rows // r,),
        in_specs=[
            pl.BlockSpec((1, D), lambda i: (0, 0)),
            pl.BlockSpec(memory_space=pltpu.MemorySpace.SMEM),
            pl.BlockSpec((r, D), lambda i: (i, 0)),
        ],
        out_specs=pl.BlockSpec((r, D), lambda i: (i, 0)),
        out_shape=jax.ShapeDtypeStruct((rows, D), jnp.float32),
    )(imp2, coeffs, x2)
    return out.reshape(B, T, D)


# speedup vs baseline: 1.1201x; 1.1201x over previous
import jax
import jax.numpy as jnp
from jax.experimental import pallas as pl
from jax.experimental.pallas import tpu as pltpu


_ROWS_PER_BLOCK = 2048


def _poly_body(imp_ref, c_ref, x_ref, o_ref, abg_ref):
    @pl.when(pl.program_id(0) == 0)
    def _():
        m = jax.nn.sigmoid(imp_ref[...])
        m = m / (jnp.mean(m) + 1e-6)
        m2 = m * m
        abg_ref[0] = c_ref[0] * m
        abg_ref[1] = c_ref[1] * m2
        abg_ref[2] = c_ref[2] * (m2 * m)

    a = abg_ref[0]
    b = abg_ref[1]
    g = abg_ref[2]
    x = x_ref[...]
    o_ref[...] = x * (a + x * (b + x * g))


def kernel(x, coeffs, importance):
    B, T, D = x.shape
    rows = B * T
    x2 = x.reshape(rows, D)
    imp2 = importance.reshape(1, D)
    r = _ROWS_PER_BLOCK
    out = pl.pallas_call(
        _poly_body,
        grid=(rows // r,),
        in_specs=[
            pl.BlockSpec((1, D), lambda i: (0, 0)),
            pl.BlockSpec(memory_space=pltpu.MemorySpace.SMEM),
            pl.BlockSpec((r, D), lambda i: (i, 0)),
        ],
        out_specs=pl.BlockSpec((r, D), lambda i: (i, 0)),
        out_shape=jax.ShapeDtypeStruct((rows, D), jnp.float32),
        scratch_shapes=[pltpu.VMEM((3, 1, D), jnp.float32)],
    )(imp2, coeffs, x2)
    return out.reshape(B, T, D)
